# Initial kernel scaffold; baseline (speedup 1.0000x reference)
import jax
import jax.numpy as jnp
from jax.experimental import pallas as pl


def kernel(x, weight):
    raise NotImplementedError("write your pallas kernel here")


# speedup vs baseline: 1.5758x; 1.5758x over previous
import jax
import jax.numpy as jnp
from jax import lax
from jax.experimental import pallas as pl
from jax.experimental.pallas import tpu as pltpu
from jax.experimental.pallas import tpu_sc as plsc

B = 4096
DIN = 8192
DOUT = 2048
NC, NS, LANES = 2, 16, 16
NW = NC * NS
RPW = B // NW
R = 4
NBUF = 2
G = RPW // R
CHUNKS = DOUT // LANES


def _body(x_hbm, cidx_hbm, out_hbm, idx_v, in0_v, in1_v, out0_v, out1_v,
          in_sem0, in_sem1, out_sem0, out_sem1):
    in_bufs = (in0_v, in1_v)
    out_bufs = (out0_v, out1_v)
    in_sems = (in_sem0, in_sem1)
    out_sems = (out_sem0, out_sem1)
    c = lax.axis_index("c")
    s = lax.axis_index("s")
    wid = s * NC + c
    row0 = wid * RPW

    pltpu.sync_copy(cidx_hbm, idx_v)

    for b in range(NBUF):
        pltpu.async_copy(x_hbm.at[pl.ds(row0 + b * R, R)],
                         in_bufs[b], in_sems[b])

    @pl.loop(0, G, step=NBUF)
    def _outer(g):
        for b in range(NBUF):
            i = g + b
            pltpu.make_async_copy(x_hbm.at[pl.ds(0, R)], in_bufs[b],
                                  in_sems[b]).wait()

            @pl.when(i >= NBUF)
            def _():
                pltpu.make_async_copy(out_bufs[b],
                                      out_hbm.at[pl.ds(0, R)],
                                      out_sems[b]).wait()

            @pl.loop(0, CHUNKS, unroll=4)
            def _inner(o):
                iv = idx_v[pl.ds(o * LANES, LANES)]
                for r in range(R):
                    rv = jnp.full((LANES,), r, jnp.int32)
                    vals = plsc.load_gather(in_bufs[b], [rv, iv])
                    out_bufs[b][r, pl.ds(o * LANES, LANES)] = vals

            pltpu.async_copy(out_bufs[b],
                             out_hbm.at[pl.ds(row0 + i * R, R)],
                             out_sems[b])

            @pl.when(i + NBUF < G)
            def _():
                pltpu.async_copy(
                    x_hbm.at[pl.ds(row0 + (i + NBUF) * R, R)],
                    in_bufs[b], in_sems[b])

    for b in range(NBUF):
        pltpu.make_async_copy(out_bufs[b], out_hbm.at[pl.ds(0, R)],
                              out_sems[b]).wait()


def kernel(x, weight):
    cidx = (jnp.arange(DIN // 128, dtype=jnp.int32)[:, None] * 128
            + weight.astype(jnp.int32)[None, :]).reshape(-1)
    cidx = jnp.clip(cidx, 0, DIN - 1)

    mesh = plsc.VectorSubcoreMesh(core_axis_name="c", subcore_axis_name="s")
    f = pl.kernel(
        _body,
        out_type=jax.ShapeDtypeStruct((B, DOUT), jnp.float32),
        mesh=mesh,
        compiler_params=pltpu.CompilerParams(needs_layout_passes=False),
        scratch_types=[
            pltpu.VMEM((DOUT,), jnp.int32),
            pltpu.VMEM((R, DIN), jnp.float32),
            pltpu.VMEM((R, DIN), jnp.float32),
            pltpu.VMEM((R, DOUT), jnp.float32),
            pltpu.VMEM((R, DOUT), jnp.float32),
            pltpu.SemaphoreType.DMA,
            pltpu.SemaphoreType.DMA,
            pltpu.SemaphoreType.DMA,
            pltpu.SemaphoreType.DMA,
        ],
    )
    return f(x, cidx)


# speedup vs baseline: 2.2662x; 1.4381x over previous
import jax
import jax.numpy as jnp
from jax import lax
from jax.experimental import pallas as pl
from jax.experimental.pallas import tpu as pltpu
from jax.experimental.pallas import tpu_sc as plsc

B = 4096
DIN = 8192
DOUT = 2048
NC, NS, LANES = 2, 16, 16
NW = NC * NS
RPW = B // NW
R = 4
NBUF = 2
G = RPW // R
CHUNKS = DOUT // LANES


def _body(x_hbm, cidx_hbm, out_hbm, idx_v, in0_v, in1_v, out0_v, out1_v,
          in_sem0, in_sem1, out_sem0, out_sem1):
    in_bufs = (in0_v, in1_v)
    out_bufs = (out0_v, out1_v)
    in_sems = (in_sem0, in_sem1)
    out_sems = (out_sem0, out_sem1)
    c = lax.axis_index("c")
    s = lax.axis_index("s")
    wid = s * NC + c
    row0 = wid * RPW

    pltpu.sync_copy(cidx_hbm, idx_v)

    def start_in(block, b):
        for r in range(R):
            pltpu.async_copy(x_hbm.at[row0 + block * R + r],
                             in_bufs[b].at[pl.ds(r * DIN, DIN)], in_sems[b])

    def wait_in(b):
        for r in range(R):
            pltpu.make_async_copy(x_hbm.at[0],
                                  in_bufs[b].at[pl.ds(0, DIN)],
                                  in_sems[b]).wait()

    def start_out(block, b):
        for r in range(R):
            pltpu.async_copy(out_bufs[b].at[pl.ds(r * DOUT, DOUT)],
                             out_hbm.at[row0 + block * R + r], out_sems[b])

    def wait_out(b):
        for r in range(R):
            pltpu.make_async_copy(out_bufs[b].at[pl.ds(0, DOUT)],
                                  out_hbm.at[0], out_sems[b]).wait()

    for b in range(NBUF):
        start_in(b, b)

    @pl.loop(0, G, step=NBUF)
    def _outer(g):
        for b in range(NBUF):
            i = g + b
            wait_in(b)

            @pl.when(i >= NBUF)
            def _():
                wait_out(b)

            @plsc.parallel_loop(0, CHUNKS, unroll=4)
            def _inner(o):
                iv = idx_v[pl.ds(o * LANES, LANES)]
                for r in range(R):
                    vals = plsc.load_gather(
                        in_bufs[b].at[pl.ds(r * DIN, DIN)], [iv])
                    out_bufs[b][pl.ds(r * DOUT + o * LANES, LANES)] = vals

            start_out(i, b)

            @pl.when(i + NBUF < G)
            def _():
                start_in(i + NBUF, b)

    for b in range(NBUF):
        wait_out(b)


def kernel(x, weight):
    cidx = (jnp.arange(DIN // 128, dtype=jnp.int32)[:, None] * 128
            + weight.astype(jnp.int32)[None, :]).reshape(-1)
    cidx = jnp.clip(cidx, 0, DIN - 1)

    mesh = plsc.VectorSubcoreMesh(core_axis_name="c", subcore_axis_name="s")
    f = pl.kernel(
        _body,
        out_type=jax.ShapeDtypeStruct((B, DOUT), jnp.float32),
        mesh=mesh,
        compiler_params=pltpu.CompilerParams(needs_layout_passes=False),
        scratch_types=[
            pltpu.VMEM((DOUT,), jnp.int32),
            pltpu.VMEM((R * DIN,), jnp.float32),
            pltpu.VMEM((R * DIN,), jnp.float32),
            pltpu.VMEM((R * DOUT,), jnp.float32),
            pltpu.VMEM((R * DOUT,), jnp.float32),
            pltpu.SemaphoreType.DMA,
            pltpu.SemaphoreType.DMA,
            pltpu.SemaphoreType.DMA,
            pltpu.SemaphoreType.DMA,
        ],
    )
    return f(x, cidx)


# speedup vs baseline: 2.2701x; 1.0017x over previous
import jax
import jax.numpy as jnp
from jax import lax
from jax.experimental import pallas as pl
from jax.experimental.pallas import tpu as pltpu
from jax.experimental.pallas import tpu_sc as plsc

B = 4096
DIN = 8192
DOUT = 2048
NC, NS, LANES = 2, 16, 16
NW = NC * NS
RPW = B // NW
R = 4
NBUF = 2
G = RPW // R
CHUNKS = DOUT // LANES


def _body(x_hbm, cidx_hbm, out_hbm, idx_v, in0_v, in1_v, out0_v, out1_v,
          in_sem0, in_sem1, out_sem0, out_sem1):
    in_bufs = (in0_v, in1_v)
    out_bufs = (out0_v, out1_v)
    in_sems = (in_sem0, in_sem1)
    out_sems = (out_sem0, out_sem1)
    c = lax.axis_index("c")
    s = lax.axis_index("s")
    wid = s * NC + c
    row0 = wid * RPW

    pltpu.sync_copy(cidx_hbm, idx_v)

    def start_in(block, b):
        pltpu.async_copy(x_hbm.at[pl.ds(row0 + block * R, R)], in_bufs[b],
                         in_sems[b])

    def wait_in(b):
        pltpu.make_async_copy(x_hbm.at[pl.ds(0, R)], in_bufs[b],
                              in_sems[b]).wait()

    def start_out(block, b):
        pltpu.async_copy(out_bufs[b], out_hbm.at[pl.ds(row0 + block * R, R)],
                         out_sems[b])

    def wait_out(b):
        pltpu.make_async_copy(out_bufs[b], out_hbm.at[pl.ds(0, R)],
                              out_sems[b]).wait()

    for b in range(NBUF):
        start_in(b, b)

    @pl.loop(0, G, step=NBUF)
    def _outer(g):
        for b in range(NBUF):
            i = g + b
            wait_in(b)

            @pl.when(i >= NBUF)
            def _():
                wait_out(b)

            @plsc.parallel_loop(0, CHUNKS, unroll=4)
            def _inner(o):
                iv = idx_v[pl.ds(o * LANES, LANES)]
                for r in range(R):
                    rv = jnp.full((LANES,), r, jnp.int32)
                    vals = plsc.load_gather(in_bufs[b], [rv, iv])
                    out_bufs[b][r, pl.ds(o * LANES, LANES)] = vals

            start_out(i, b)

            @pl.when(i + NBUF < G)
            def _():
                start_in(i + NBUF, b)

    for b in range(NBUF):
        wait_out(b)


def kernel(x, weight):
    cidx = (jnp.arange(DIN // 128, dtype=jnp.int32)[:, None] * 128
            + weight.astype(jnp.int32)[None, :]).reshape(-1)
    cidx = jnp.clip(cidx, 0, DIN - 1)

    mesh = plsc.VectorSubcoreMesh(core_axis_name="c", subcore_axis_name="s")
    f = pl.kernel(
        _body,
        out_type=jax.ShapeDtypeStruct((B, DOUT), jnp.float32),
        mesh=mesh,
        compiler_params=pltpu.CompilerParams(needs_layout_passes=False),
        scratch_types=[
            pltpu.VMEM((DOUT,), jnp.int32),
            pltpu.VMEM((R, DIN), jnp.float32),
            pltpu.VMEM((R, DIN), jnp.float32),
            pltpu.VMEM((R, DOUT), jnp.float32),
            pltpu.VMEM((R, DOUT), jnp.float32),
            pltpu.SemaphoreType.DMA,
            pltpu.SemaphoreType.DMA,
            pltpu.SemaphoreType.DMA,
            pltpu.SemaphoreType.DMA,
        ],
    )
    return f(x, cidx)


# speedup vs baseline: 2.3281x; 1.0256x over previous
import jax
import jax.numpy as jnp
from jax import lax
from jax.experimental import pallas as pl
from jax.experimental.pallas import tpu as pltpu
from jax.experimental.pallas import tpu_sc as plsc

B = 4096
DIN = 8192
DOUT = 2048
NC, NS, LANES = 2, 16, 16
HIN = DIN // NC
HOUT = DOUT // NC
R = 8
NBUF = 2
G = B // (NS * R)
CHUNKS = HOUT // LANES


def _body(x_hbm, cidx_hbm, out_hbm, idx_v, in0_v, in1_v, out0_v, out1_v,
          in_sem0, in_sem1, out_sem0, out_sem1):
    in_bufs = (in0_v, in1_v)
    out_bufs = (out0_v, out1_v)
    in_sems = (in_sem0, in_sem1)
    out_sems = (out_sem0, out_sem1)
    h = lax.axis_index("c")
    s = lax.axis_index("s")
    row0 = s * (G * R)

    pltpu.sync_copy(cidx_hbm.at[h], idx_v)

    def start_in(block, b):
        pltpu.async_copy(
            x_hbm.at[pl.ds(row0 + block * R, R), pl.ds(h * HIN, HIN)],
            in_bufs[b], in_sems[b])

    def wait_in(b):
        pltpu.make_async_copy(
            x_hbm.at[pl.ds(0, R), pl.ds(0, HIN)], in_bufs[b],
            in_sems[b]).wait()

    def start_out(block, b):
        pltpu.async_copy(
            out_bufs[b],
            out_hbm.at[pl.ds(row0 + block * R, R), pl.ds(h * HOUT, HOUT)],
            out_sems[b])

    def wait_out(b):
        pltpu.make_async_copy(
            out_bufs[b], out_hbm.at[pl.ds(0, R), pl.ds(0, HOUT)],
            out_sems[b]).wait()

    for b in range(NBUF):
        start_in(b, b)

    @pl.loop(0, G, step=NBUF)
    def _outer(g):
        for b in range(NBUF):
            i = g + b
            wait_in(b)

            @pl.when(i >= NBUF)
            def _():
                wait_out(b)

            @plsc.parallel_loop(0, CHUNKS, unroll=4)
            def _inner(o):
                iv = idx_v[pl.ds(o * LANES, LANES)]
                for r in range(R):
                    rv = jnp.full((LANES,), r, jnp.int32)
                    vals = plsc.load_gather(in_bufs[b], [rv, iv])
                    out_bufs[b][r, pl.ds(o * LANES, LANES)] = vals

            start_out(i, b)

            @pl.when(i + NBUF < G)
            def _():
                start_in(i + NBUF, b)

    for b in range(NBUF):
        wait_out(b)


def kernel(x, weight):
    cidx = (jnp.arange(DIN // 128, dtype=jnp.int32)[:, None] * 128
            + weight.astype(jnp.int32)[None, :]).reshape(-1)
    cidx = jnp.clip(cidx, 0, DIN - 1)
    halves = []
    for hh in range(NC):
        lo, hi = hh * HIN, (hh + 1) * HIN - 1
        halves.append(jnp.clip(cidx[hh * HOUT:(hh + 1) * HOUT], lo, hi) - lo)
    cidx2 = jnp.stack(halves)

    mesh = plsc.VectorSubcoreMesh(core_axis_name="c", subcore_axis_name="s")
    f = pl.kernel(
        _body,
        out_type=jax.ShapeDtypeStruct((B, DOUT), jnp.float32),
        mesh=mesh,
        compiler_params=pltpu.CompilerParams(needs_layout_passes=False),
        scratch_types=[
            pltpu.VMEM((HOUT,), jnp.int32),
            pltpu.VMEM((R, HIN), jnp.float32),
            pltpu.VMEM((R, HIN), jnp.float32),
            pltpu.VMEM((R, HOUT), jnp.float32),
            pltpu.VMEM((R, HOUT), jnp.float32),
            pltpu.SemaphoreType.DMA,
            pltpu.SemaphoreType.DMA,
            pltpu.SemaphoreType.DMA,
            pltpu.SemaphoreType.DMA,
        ],
    )
    return f(x, cidx2)
